# baseline (speedup 1.0000x reference)
import dataclasses
import functools

import jax
import jax.numpy as jnp
from jax import lax
from jax.experimental import pallas as pl
from jax.experimental.pallas import tpu as pltpu
from jax.experimental.pallas import tpu_sc as plsc

N = 50000
E = 800000
HIST = 12
IN_DIM = 2
OUT_CH = 2
H = 3
F_IN = HIST * IN_DIM
C = HIST * OUT_CH
HID = 64
PRED = 6

NB_TC = 16
NPAD = 50048
BLK = NPAD // NB_TC
NPF = 25088
NBF = 32
BLKF = NPF // NBF
ROWW = 80
ADW = 16
QUART = N // 4
NQ = 4
QACC = 12544
WPR = QACC // 16

NCORE = 2
NSUB = 16
NPASS = 2
EPS = E // NSUB
ROUNDS = 25
ECH = EPS // ROUNDS
BBLK = 128
CSIZE = ECH + BBLK
WPITCH = 81


def _prelude_body(xi_ref, mx_ref, cx_ref, md_ref, xpe_ref, ade_ref):
    xi = xi_ref[...]
    xpe_ref[...] = lax.dot_general(
        xi, mx_ref[...], (((1,), (0,)), ((), ())),
        preferred_element_type=jnp.float32,
        precision=lax.Precision.HIGHEST) + cx_ref[...]
    ade_ref[...] = lax.dot_general(
        xi, md_ref[...], (((1,), (0,)), ((), ())),
        preferred_element_type=jnp.float32,
        precision=lax.Precision.HIGHEST)


def _prelude(xi, mx, cx, md):
    return pl.pallas_call(
        _prelude_body,
        grid=(NB_TC,),
        in_specs=[
            pl.BlockSpec((BLK, F_IN), lambda i: (i, 0)),
            pl.BlockSpec((F_IN, ROWW), lambda i: (0, 0)),
            pl.BlockSpec((1, ROWW), lambda i: (0, 0)),
            pl.BlockSpec((F_IN, ADW), lambda i: (0, 0)),
        ],
        out_specs=[
            pl.BlockSpec((BLK, ROWW), lambda i: (i, 0)),
            pl.BlockSpec((BLK, ADW), lambda i: (i, 0)),
        ],
        out_shape=[
            jax.ShapeDtypeStruct((NPAD, ROWW), jnp.float32),
            jax.ShapeDtypeStruct((NPAD, ADW), jnp.float32),
        ],
    )(xi, mx, cx, md)


@functools.cache
def _build_gat_sc(pass_idx):
    mesh = plsc.VectorSubcoreMesh(core_axis_name="c", subcore_axis_name="s",
                                  num_cores=NCORE, num_subcores=NSUB)
    cp = pltpu.CompilerParams(needs_layout_passes=False,
                              use_tc_tiling_on_sc=False)
    return pl.kernel(
        functools.partial(_gat_sc_body, pass_idx),
        out_type=jax.ShapeDtypeStruct((NCORE, QACC, ROWW), jnp.float32),
        mesh=mesh,
        scratch_types=[
            pltpu.VMEM((ECH,), jnp.int32),
            pltpu.VMEM((ECH,), jnp.int32),
            pltpu.VMEM((CSIZE,), jnp.int32),
            pltpu.VMEM((CSIZE,), jnp.int32),
            pltpu.VMEM((BBLK, ROWW), jnp.float32),
            pltpu.VMEM((BBLK, ROWW), jnp.float32),
            pltpu.VMEM((BBLK, ROWW), jnp.float32),
            pltpu.VMEM((BBLK * WPITCH,), jnp.float32),
            pltpu.VMEM((BBLK, ADW), jnp.float32),
            pltpu.VMEM((BBLK, ADW), jnp.float32),
            pltpu.VMEM((BBLK, ADW), jnp.float32),
            pltpu.VMEM((BBLK,), jnp.int32),
            pltpu.VMEM((BBLK,), jnp.int32),
            pltpu.VMEM((BBLK,), jnp.int32),
            pltpu.VMEM_SHARED((QACC, ROWW), jnp.float32),
            pltpu.SemaphoreType.DMA,
            pltpu.SemaphoreType.DMA,
            pltpu.SemaphoreType.DMA,
            pltpu.SemaphoreType.DMA,
            pltpu.SemaphoreType.DMA,
            pltpu.SemaphoreType.DMA,
        ],
        compiler_params=cp,
    )


def _gat_sc_body(pass_idx, src_hbm, dst_hbm, xpe_hbm, ade_hbm, out_hbm,
                 sbuf, dbuf, csrc, cdst, rows0, rows1, rows2, wbuf,
                 adv0, adv1, adv2, lidx0, lidx1, lidx2, acc,
                 gsem0, gsem1, gsem2, ssem0, ssem1, ssem2):
    c = lax.axis_index("c")
    s = lax.axis_index("s")
    iota = lax.iota(jnp.int32, 16)
    zf = jnp.zeros((16,), jnp.float32)

    @pl.loop(0, BBLK * WPITCH // 16)
    def _zw(j):
        wbuf[pl.ds(j * 16, 16)] = zf

    if True:
        q = c * NPASS + pass_idx
        lo = q * QUART

        @pl.loop(0, BBLK)
        def _zr(e):
            for kk in range(ROWW // 16):
                rows0[e, pl.ds(kk * 16, 16)] = zf

        for j in range(WPR // 112):
            pltpu.sync_copy(rows0.at[pl.ds(0, 112)],
                            acc.at[pl.ds(s * WPR + j * 112, 112)])
        plsc.subcore_barrier()

        @pl.loop(0, ROUNDS)
        def _round(r):
            base = s * EPS + r * ECH
            cp0 = pltpu.async_copy(src_hbm.at[pl.ds(base, ECH)], sbuf, gsem0)
            cp1 = pltpu.async_copy(dst_hbm.at[pl.ds(base, ECH)], dbuf, gsem1)
            cp0.wait()
            cp1.wait()

            def _grpA(g, cnt):
                dg = dbuf[pl.ds(g * 16, 16)]
                sg = sbuf[pl.ds(g * 16, 16)]
                m = (dg >= lo) & (dg < lo + QUART)
                mi = jnp.where(m, 1, 0)
                pos = cnt + plsc.cumsum(mi) - 1
                plsc.store_scatter(csrc, [pos], sg, mask=m)
                plsc.store_scatter(cdst, [pos], dg, mask=m)
                return cnt + jnp.sum(mi)

            k = lax.fori_loop(0, ECH // 16, _grpA, jnp.int32(0))

            @pl.loop(0, BBLK // 16)
            def _pad(j):
                pidx = k + j * 16 + iota
                plsc.store_scatter(csrc, [pidx], N + iota)
                plsc.store_scatter(cdst, [pidx],
                                   lo + QUART + ((iota + j) & 31))

            nb = (k + BBLK - 1) // BBLK
            nbt = (nb + 2) // 3

            sets = ((rows0, adv0, lidx0, gsem0, ssem0),
                    (rows1, adv1, lidx1, gsem1, ssem1),
                    (rows2, adv2, lidx2, gsem2, ssem2))

            def _gissue(off, st):
                rbuf, abuf, _, gsem, _ = st
                pltpu.async_copy(xpe_hbm.at[csrc.at[pl.ds(off, BBLK)]],
                                 rbuf, gsem)
                pltpu.async_copy(ade_hbm.at[cdst.at[pl.ds(off, BBLK)]],
                                 abuf, gsem)

            def _gwait(st):
                rbuf, abuf, _, gsem, _ = st
                pltpu.make_async_copy(
                    xpe_hbm.at[csrc.at[pl.ds(0, BBLK)]], rbuf, gsem).wait()
                pltpu.make_async_copy(
                    ade_hbm.at[cdst.at[pl.ds(0, BBLK)]], abuf, gsem).wait()

            def _swait(st):
                rbuf, _, lbuf, _, ssem = st
                pltpu.make_async_copy(rbuf, acc.at[lbuf], ssem).wait()

            def _compute(off, st):
                rbuf, abuf, lbuf, _, ssem = st

                @pl.loop(0, BBLK // 16, unroll=2)
                def _grp(g):
                    e16 = iota + g * 16
                    dg = cdst[pl.ds(off + g * 16, 16)]
                    lbuf[pl.ds(g * 16, 16)] = dg - lo
                    wpos = e16 * WPITCH
                    for h in range(H):
                        a1 = plsc.load_gather(
                            rbuf, [e16, jnp.full((16,), 72 + h, jnp.int32)])
                        a2 = plsc.load_gather(
                            abuf, [e16, jnp.full((16,), h, jnp.int32)])
                        al = a1 + a2
                        al = jnp.where(al >= 0.0, al, al * 0.2)
                        wv = jnp.exp(al)
                        for cc in range(C):
                            plsc.store_scatter(wbuf, [wpos + (C * h + cc)], wv)
                        plsc.store_scatter(wbuf, [wpos + (75 + h)], wv)

                @pl.loop(0, BBLK, unroll=4)
                def _mul(e):
                    for kk in range(ROWW // 16):
                        rbuf[e, pl.ds(kk * 16, 16)] = (
                            rbuf[e, pl.ds(kk * 16, 16)]
                            * wbuf[pl.ds(e * WPITCH + kk * 16, 16)])

                pltpu.async_copy(rbuf, acc.at[lbuf], ssem, add=True)

            @pl.when(nb > 0)
            def _p0():
                _gissue(0, sets[0])

            @pl.when(nb > 1)
            def _p1():
                _gissue(BBLK, sets[1])

            @pl.loop(0, nbt)
            def _blk3(b3):
                for j in range(3):
                    st = sets[j]
                    b = 3 * b3 + j

                    @pl.when(b < nb)
                    def _do():
                        _gwait(st)
                        _compute(b * BBLK, st)

                        nxt = b + 2

                        @pl.when(nxt < nb)
                        def _issue_next():
                            stn = sets[(j + 2) % 3]
                            if j == 0:
                                @pl.when(b3 >= 1)
                                def _w():
                                    _swait(stn)
                            else:
                                _swait(stn)
                            _gissue(nxt * BBLK, stn)

            for j in range(3):
                @pl.when(nb > j)
                def _dr():
                    _swait(sets[j])

        plsc.subcore_barrier()
        pltpu.sync_copy(acc.at[pl.ds(s * WPR, WPR)],
                        out_hbm.at[c, pl.ds(s * WPR, WPR)])


def _finale_body(acc_ref, xpe_ref, ade_ref, gatb_ref, wir_ref, wiz_ref,
                 win_ref, whr_ref, whz_ref, whn_ref, br_ref, bz_ref, bn_ref,
                 hbr_ref, hbz_ref, hbn_ref, p1w_ref, p1b_ref, p2wt_ref,
                 p2b_ref, out_ref):
    accb = acc_ref[...]
    xpe = xpe_ref[...]
    ade = ade_ref[...]
    go = jnp.zeros((BLKF, C), jnp.float32)
    for h in range(H):
        al = xpe[:, 72 + h:73 + h] + ade[:, h:h + 1]
        ws = jnp.exp(jnp.where(al >= 0.0, al, al * 0.2))
        num = accb[:, C * h:C * h + C] + ws * xpe[:, C * h:C * h + C]
        den = accb[:, 75 + h:76 + h] + ws
        go = go + num / (den + 1e-16)
    go = go * (1.0 / 3.0) + gatb_ref[...]

    wir = wir_ref[...]
    wiz = wiz_ref[...]
    win = win_ref[...]
    whr = whr_ref[...]
    whz = whz_ref[...]
    whn = whn_ref[...]
    p1w = p1w_ref[...]

    def mm(a, b):
        return lax.dot_general(a, b, (((1,), (0,)), ((), ())),
                               preferred_element_type=jnp.float32,
                               precision=lax.Precision.DEFAULT)

    hstate = jnp.zeros((BLKF, HID), jnp.float32)
    out6 = jnp.zeros((BLKF, PRED), jnp.float32)
    for t in range(HIST):
        x0 = go[:, 2 * t:2 * t + 1]
        x1 = go[:, 2 * t + 1:2 * t + 2]
        gir = x0 * wir[0:1, :] + x1 * wir[1:2, :] + br_ref[...]
        giz = x0 * wiz[0:1, :] + x1 * wiz[1:2, :] + bz_ref[...]
        gin = x0 * win[0:1, :] + x1 * win[1:2, :] + bn_ref[...]
        r = 0.5 * jnp.tanh(0.5 * (gir + mm(hstate, whr) + hbr_ref[...])) + 0.5
        z = 0.5 * jnp.tanh(0.5 * (giz + mm(hstate, whz) + hbz_ref[...])) + 0.5
        cc = jnp.tanh(gin + r * (mm(hstate, whn) + hbn_ref[...]))
        hstate = cc + z * (hstate - cc)
        ot = jnp.sum(hstate * p1w, axis=1, keepdims=True) + p1b_ref[...]
        out6 = out6 + ot * p2wt_ref[...][t:t + 1, :]
    out_ref[...] = out6 + p2b_ref[...]


def _finale(accn, xpe, ade, gatb, wih_t, whh_t, b_ih, b_hh, p1w, p1b,
            p2wt, p2b):
    def full(shape):
        return pl.BlockSpec(shape, lambda i: tuple(0 for _ in shape))
    gates_i = [wih_t[:, g * HID:(g + 1) * HID] for g in range(3)]
    gates_h = [whh_t[:, g * HID:(g + 1) * HID] for g in range(3)]
    bi = [b_ih[:, g * HID:(g + 1) * HID] for g in range(3)]
    bh = [b_hh[:, g * HID:(g + 1) * HID] for g in range(3)]
    return pl.pallas_call(
        _finale_body,
        grid=(NBF,),
        in_specs=[
            pl.BlockSpec((BLKF, ROWW), lambda i: (i, 0)),
            pl.BlockSpec((BLKF, ROWW), lambda i: (i, 0)),
            pl.BlockSpec((BLKF, ADW), lambda i: (i, 0)),
            full((1, C)),
            full((IN_DIM, HID)), full((IN_DIM, HID)), full((IN_DIM, HID)),
            full((HID, HID)), full((HID, HID)), full((HID, HID)),
            full((1, HID)), full((1, HID)), full((1, HID)),
            full((1, HID)), full((1, HID)), full((1, HID)),
            full((1, HID)),
            full((1, 1)),
            full((HIST, PRED)),
            full((1, PRED)),
        ],
        out_specs=pl.BlockSpec((BLKF, PRED), lambda i: (i, 0)),
        out_shape=jax.ShapeDtypeStruct((NPF, PRED), jnp.float32),
    )(accn, xpe, ade, gatb, *gates_i, *gates_h, *bi, *bh, p1w, p1b,
      p2wt, p2b)


def kernel(x, edge_index, gat_w, att_src, att_dst, gat_b, w_ih, w_hh,
           b_ih, b_hh, p1_w, p1_b, p2_w, p2_b):
    xi = x.reshape(N, F_IN)
    xi = jnp.pad(xi, ((0, NPAD - N), (0, 0)))
    gwt = gat_w.T
    asr_m = jnp.zeros((H * C, H), jnp.float32)
    adr_m = jnp.zeros((H * C, H), jnp.float32)
    for h in range(H):
        asr_m = asr_m.at[C * h:C * h + C, h].set(att_src.reshape(H, C)[h])
        adr_m = adr_m.at[C * h:C * h + C, h].set(att_dst.reshape(H, C)[h])
    mx = jnp.concatenate(
        [gwt, gwt @ asr_m, jnp.zeros((F_IN, ROWW - 75), jnp.float32)], axis=1)
    cx = jnp.zeros((1, ROWW), jnp.float32).at[0, 75:78].set(1.0)
    md = jnp.concatenate(
        [gwt @ adr_m, jnp.zeros((F_IN, ADW - H), jnp.float32)], axis=1)
    xpe, ade = _prelude(xi, mx, cx, md)
    src, dst = edge_index[0], edge_index[1]
    wargs = (gat_b.reshape(1, C), w_ih.T, w_hh.T,
             b_ih.reshape(1, 3 * HID), b_hh.reshape(1, 3 * HID),
             p1_w, p1_b.reshape(1, 1), p2_w.T, p2_b.reshape(1, PRED))

    accA = _build_gat_sc(0)(src, dst, xpe, ade)
    accB = _build_gat_sc(1)(src, dst, xpe, ade)

    def half(acc2, q0, q1):
        accn = jnp.concatenate([acc2[0, :QUART], acc2[1, :QUART]], axis=0)
        accn = jnp.pad(accn, ((0, NPF - 2 * QUART), (0, 0)))
        xh = jnp.concatenate([xpe[q0 * QUART:(q0 + 1) * QUART],
                              xpe[q1 * QUART:(q1 + 1) * QUART]], axis=0)
        xh = jnp.pad(xh, ((0, NPF - 2 * QUART), (0, 0)))
        ah = jnp.concatenate([ade[q0 * QUART:(q0 + 1) * QUART],
                              ade[q1 * QUART:(q1 + 1) * QUART]], axis=0)
        ah = jnp.pad(ah, ((0, NPF - 2 * QUART), (0, 0)))
        return _finale(accn, xh, ah, *wargs)

    outA = half(accA, 0, 2)
    outB = half(accB, 1, 3)
    out = jnp.concatenate([outA[:QUART], outB[:QUART],
                           outA[QUART:2 * QUART], outB[QUART:2 * QUART]],
                          axis=0)
    return jnp.transpose(out.reshape(1, N, PRED), (0, 2, 1))


# speedup vs baseline: 1.0037x; 1.0037x over previous
import dataclasses
import functools

import jax
import jax.numpy as jnp
from jax import lax
from jax.experimental import pallas as pl
from jax.experimental.pallas import tpu as pltpu
from jax.experimental.pallas import tpu_sc as plsc

N = 50000
E = 800000
HIST = 12
IN_DIM = 2
OUT_CH = 2
H = 3
F_IN = HIST * IN_DIM
C = HIST * OUT_CH
HID = 64
PRED = 6

NB_TC = 16
NPAD = 50048
BLK = NPAD // NB_TC
NPF = 25088
NBF = 16
BLKF = NPF // NBF
ROWW = 80
ADW = 16
QUART = N // 4
NQ = 4
QACC = 12544
WPR = QACC // 16

NCORE = 2
NSUB = 16
NPASS = 2
EPS = E // NSUB
ROUNDS = 25
ECH = EPS // ROUNDS
BBLK = 128
CSIZE = ECH + BBLK
WPITCH = 81


def _prelude_body(xi_ref, mx_ref, cx_ref, md_ref, xpe_ref, ade_ref):
    xi = xi_ref[...]
    xpe_ref[...] = lax.dot_general(
        xi, mx_ref[...], (((1,), (0,)), ((), ())),
        preferred_element_type=jnp.float32,
        precision=lax.Precision.HIGHEST) + cx_ref[...]
    ade_ref[...] = lax.dot_general(
        xi, md_ref[...], (((1,), (0,)), ((), ())),
        preferred_element_type=jnp.float32,
        precision=lax.Precision.HIGHEST)


def _prelude(xi, mx, cx, md):
    return pl.pallas_call(
        _prelude_body,
        grid=(NB_TC,),
        in_specs=[
            pl.BlockSpec((BLK, F_IN), lambda i: (i, 0)),
            pl.BlockSpec((F_IN, ROWW), lambda i: (0, 0)),
            pl.BlockSpec((1, ROWW), lambda i: (0, 0)),
            pl.BlockSpec((F_IN, ADW), lambda i: (0, 0)),
        ],
        out_specs=[
            pl.BlockSpec((BLK, ROWW), lambda i: (i, 0)),
            pl.BlockSpec((BLK, ADW), lambda i: (i, 0)),
        ],
        out_shape=[
            jax.ShapeDtypeStruct((NPAD, ROWW), jnp.float32),
            jax.ShapeDtypeStruct((NPAD, ADW), jnp.float32),
        ],
    )(xi, mx, cx, md)


@functools.cache
def _build_gat_sc(pass_idx):
    mesh = plsc.VectorSubcoreMesh(core_axis_name="c", subcore_axis_name="s",
                                  num_cores=NCORE, num_subcores=NSUB)
    cp = pltpu.CompilerParams(needs_layout_passes=False,
                              use_tc_tiling_on_sc=False)
    return pl.kernel(
        functools.partial(_gat_sc_body, pass_idx),
        out_type=jax.ShapeDtypeStruct((NCORE, QACC, ROWW), jnp.float32),
        mesh=mesh,
        scratch_types=[
            pltpu.VMEM((ECH,), jnp.int32),
            pltpu.VMEM((ECH,), jnp.int32),
            pltpu.VMEM((CSIZE,), jnp.int32),
            pltpu.VMEM((CSIZE,), jnp.int32),
            pltpu.VMEM((BBLK, ROWW), jnp.float32),
            pltpu.VMEM((BBLK, ROWW), jnp.float32),
            pltpu.VMEM((BBLK, ROWW), jnp.float32),
            pltpu.VMEM((BBLK * WPITCH,), jnp.float32),
            pltpu.VMEM((BBLK, ADW), jnp.float32),
            pltpu.VMEM((BBLK, ADW), jnp.float32),
            pltpu.VMEM((BBLK, ADW), jnp.float32),
            pltpu.VMEM((BBLK,), jnp.int32),
            pltpu.VMEM((BBLK,), jnp.int32),
            pltpu.VMEM((BBLK,), jnp.int32),
            pltpu.VMEM_SHARED((QACC, ROWW), jnp.float32),
            pltpu.SemaphoreType.DMA,
            pltpu.SemaphoreType.DMA,
            pltpu.SemaphoreType.DMA,
            pltpu.SemaphoreType.DMA,
            pltpu.SemaphoreType.DMA,
            pltpu.SemaphoreType.DMA,
        ],
        compiler_params=cp,
    )


def _gat_sc_body(pass_idx, src_hbm, dst_hbm, xpe_hbm, ade_hbm, out_hbm,
                 sbuf, dbuf, csrc, cdst, rows0, rows1, rows2, wbuf,
                 adv0, adv1, adv2, lidx0, lidx1, lidx2, acc,
                 gsem0, gsem1, gsem2, ssem0, ssem1, ssem2):
    c = lax.axis_index("c")
    s = lax.axis_index("s")
    iota = lax.iota(jnp.int32, 16)
    zf = jnp.zeros((16,), jnp.float32)

    @pl.loop(0, BBLK * WPITCH // 16)
    def _zw(j):
        wbuf[pl.ds(j * 16, 16)] = zf

    if True:
        q = c * NPASS + pass_idx
        lo = q * QUART

        @pl.loop(0, BBLK)
        def _zr(e):
            for kk in range(ROWW // 16):
                rows0[e, pl.ds(kk * 16, 16)] = zf

        for j in range(WPR // 112):
            pltpu.sync_copy(rows0.at[pl.ds(0, 112)],
                            acc.at[pl.ds(s * WPR + j * 112, 112)])
        plsc.subcore_barrier()

        @pl.loop(0, ROUNDS)
        def _round(r):
            base = s * EPS + r * ECH
            cp0 = pltpu.async_copy(src_hbm.at[pl.ds(base, ECH)], sbuf, gsem0)
            cp1 = pltpu.async_copy(dst_hbm.at[pl.ds(base, ECH)], dbuf, gsem1)
            cp0.wait()
            cp1.wait()

            def _grpA(g, cnt):
                dg = dbuf[pl.ds(g * 16, 16)]
                sg = sbuf[pl.ds(g * 16, 16)]
                m = (dg >= lo) & (dg < lo + QUART)
                mi = jnp.where(m, 1, 0)
                pos = cnt + plsc.cumsum(mi) - 1
                plsc.store_scatter(csrc, [pos], sg, mask=m)
                plsc.store_scatter(cdst, [pos], dg, mask=m)
                return cnt + jnp.sum(mi)

            k = lax.fori_loop(0, ECH // 16, _grpA, jnp.int32(0))

            @pl.loop(0, BBLK // 16)
            def _pad(j):
                pidx = k + j * 16 + iota
                plsc.store_scatter(csrc, [pidx], N + iota)
                plsc.store_scatter(cdst, [pidx],
                                   lo + QUART + ((iota + j) & 31))

            nb = (k + BBLK - 1) // BBLK
            nbt = (nb + 2) // 3

            sets = ((rows0, adv0, lidx0, gsem0, ssem0),
                    (rows1, adv1, lidx1, gsem1, ssem1),
                    (rows2, adv2, lidx2, gsem2, ssem2))

            def _gissue(off, st):
                rbuf, abuf, _, gsem, _ = st
                pltpu.async_copy(xpe_hbm.at[csrc.at[pl.ds(off, BBLK)]],
                                 rbuf, gsem)
                pltpu.async_copy(ade_hbm.at[cdst.at[pl.ds(off, BBLK)]],
                                 abuf, gsem)

            def _gwait(st):
                rbuf, abuf, _, gsem, _ = st
                pltpu.make_async_copy(
                    xpe_hbm.at[csrc.at[pl.ds(0, BBLK)]], rbuf, gsem).wait()
                pltpu.make_async_copy(
                    ade_hbm.at[cdst.at[pl.ds(0, BBLK)]], abuf, gsem).wait()

            def _swait(st):
                rbuf, _, lbuf, _, ssem = st
                pltpu.make_async_copy(rbuf, acc.at[lbuf], ssem).wait()

            def _compute(off, st):
                rbuf, abuf, lbuf, _, ssem = st

                @pl.loop(0, BBLK // 16, unroll=2)
                def _grp(g):
                    e16 = iota + g * 16
                    dg = cdst[pl.ds(off + g * 16, 16)]
                    lbuf[pl.ds(g * 16, 16)] = dg - lo
                    wpos = e16 * WPITCH
                    for h in range(H):
                        a1 = plsc.load_gather(
                            rbuf, [e16, jnp.full((16,), 72 + h, jnp.int32)])
                        a2 = plsc.load_gather(
                            abuf, [e16, jnp.full((16,), h, jnp.int32)])
                        al = a1 + a2
                        al = jnp.where(al >= 0.0, al, al * 0.2)
                        wv = jnp.exp(al)
                        for cc in range(C):
                            plsc.store_scatter(wbuf, [wpos + (C * h + cc)], wv)
                        plsc.store_scatter(wbuf, [wpos + (75 + h)], wv)

                @pl.loop(0, BBLK, unroll=4)
                def _mul(e):
                    for kk in range(ROWW // 16):
                        rbuf[e, pl.ds(kk * 16, 16)] = (
                            rbuf[e, pl.ds(kk * 16, 16)]
                            * wbuf[pl.ds(e * WPITCH + kk * 16, 16)])

                pltpu.async_copy(rbuf, acc.at[lbuf], ssem, add=True)

            @pl.when(nb > 0)
            def _p0():
                _gissue(0, sets[0])

            @pl.when(nb > 1)
            def _p1():
                _gissue(BBLK, sets[1])

            @pl.loop(0, nbt)
            def _blk3(b3):
                for j in range(3):
                    st = sets[j]
                    b = 3 * b3 + j

                    @pl.when(b < nb)
                    def _do():
                        _gwait(st)
                        _compute(b * BBLK, st)

                        nxt = b + 2

                        @pl.when(nxt < nb)
                        def _issue_next():
                            stn = sets[(j + 2) % 3]
                            if j == 0:
                                @pl.when(b3 >= 1)
                                def _w():
                                    _swait(stn)
                            else:
                                _swait(stn)
                            _gissue(nxt * BBLK, stn)

            for j in range(3):
                @pl.when(nb > j)
                def _dr():
                    _swait(sets[j])

        plsc.subcore_barrier()
        pltpu.sync_copy(acc.at[pl.ds(s * WPR, WPR)],
                        out_hbm.at[c, pl.ds(s * WPR, WPR)])


def _finale_body(acc_ref, xpe_ref, ade_ref, gatb_ref, wir_ref, wiz_ref,
                 win_ref, whr_ref, whz_ref, whn_ref, br_ref, bz_ref, bn_ref,
                 hbr_ref, hbz_ref, hbn_ref, p1w_ref, p1b_ref, p2wt_ref,
                 p2b_ref, out_ref):
    accb = acc_ref[...]
    xpe = xpe_ref[...]
    ade = ade_ref[...]
    go = jnp.zeros((BLKF, C), jnp.float32)
    for h in range(H):
        al = xpe[:, 72 + h:73 + h] + ade[:, h:h + 1]
        ws = jnp.exp(jnp.where(al >= 0.0, al, al * 0.2))
        num = accb[:, C * h:C * h + C] + ws * xpe[:, C * h:C * h + C]
        den = accb[:, 75 + h:76 + h] + ws
        go = go + num / (den + 1e-16)
    go = go * (1.0 / 3.0) + gatb_ref[...]

    wir = wir_ref[...]
    wiz = wiz_ref[...]
    win = win_ref[...]
    whr = whr_ref[...]
    whz = whz_ref[...]
    whn = whn_ref[...]
    p1w = p1w_ref[...]

    def mm(a, b):
        return lax.dot_general(a, b, (((1,), (0,)), ((), ())),
                               preferred_element_type=jnp.float32,
                               precision=lax.Precision.DEFAULT)

    hstate = jnp.zeros((BLKF, HID), jnp.float32)
    out6 = jnp.zeros((BLKF, PRED), jnp.float32)
    for t in range(HIST):
        x0 = go[:, 2 * t:2 * t + 1]
        x1 = go[:, 2 * t + 1:2 * t + 2]
        gir = x0 * wir[0:1, :] + x1 * wir[1:2, :] + br_ref[...]
        giz = x0 * wiz[0:1, :] + x1 * wiz[1:2, :] + bz_ref[...]
        gin = x0 * win[0:1, :] + x1 * win[1:2, :] + bn_ref[...]
        r = 0.5 * jnp.tanh(0.5 * (gir + mm(hstate, whr) + hbr_ref[...])) + 0.5
        z = 0.5 * jnp.tanh(0.5 * (giz + mm(hstate, whz) + hbz_ref[...])) + 0.5
        cc = jnp.tanh(gin + r * (mm(hstate, whn) + hbn_ref[...]))
        hstate = cc + z * (hstate - cc)
        ot = jnp.sum(hstate * p1w, axis=1, keepdims=True) + p1b_ref[...]
        out6 = out6 + ot * p2wt_ref[...][t:t + 1, :]
    out_ref[...] = out6 + p2b_ref[...]


def _finale(accn, xpe, ade, gatb, wih_t, whh_t, b_ih, b_hh, p1w, p1b,
            p2wt, p2b):
    def full(shape):
        return pl.BlockSpec(shape, lambda i: tuple(0 for _ in shape))
    gates_i = [wih_t[:, g * HID:(g + 1) * HID] for g in range(3)]
    gates_h = [whh_t[:, g * HID:(g + 1) * HID] for g in range(3)]
    bi = [b_ih[:, g * HID:(g + 1) * HID] for g in range(3)]
    bh = [b_hh[:, g * HID:(g + 1) * HID] for g in range(3)]
    return pl.pallas_call(
        _finale_body,
        grid=(NBF,),
        in_specs=[
            pl.BlockSpec((BLKF, ROWW), lambda i: (i, 0)),
            pl.BlockSpec((BLKF, ROWW), lambda i: (i, 0)),
            pl.BlockSpec((BLKF, ADW), lambda i: (i, 0)),
            full((1, C)),
            full((IN_DIM, HID)), full((IN_DIM, HID)), full((IN_DIM, HID)),
            full((HID, HID)), full((HID, HID)), full((HID, HID)),
            full((1, HID)), full((1, HID)), full((1, HID)),
            full((1, HID)), full((1, HID)), full((1, HID)),
            full((1, HID)),
            full((1, 1)),
            full((HIST, PRED)),
            full((1, PRED)),
        ],
        out_specs=pl.BlockSpec((BLKF, PRED), lambda i: (i, 0)),
        out_shape=jax.ShapeDtypeStruct((NPF, PRED), jnp.float32),
    )(accn, xpe, ade, gatb, *gates_i, *gates_h, *bi, *bh, p1w, p1b,
      p2wt, p2b)


def kernel(x, edge_index, gat_w, att_src, att_dst, gat_b, w_ih, w_hh,
           b_ih, b_hh, p1_w, p1_b, p2_w, p2_b):
    xi = x.reshape(N, F_IN)
    xi = jnp.pad(xi, ((0, NPAD - N), (0, 0)))
    gwt = gat_w.T
    asr_m = jnp.zeros((H * C, H), jnp.float32)
    adr_m = jnp.zeros((H * C, H), jnp.float32)
    for h in range(H):
        asr_m = asr_m.at[C * h:C * h + C, h].set(att_src.reshape(H, C)[h])
        adr_m = adr_m.at[C * h:C * h + C, h].set(att_dst.reshape(H, C)[h])
    mx = jnp.concatenate(
        [gwt, gwt @ asr_m, jnp.zeros((F_IN, ROWW - 75), jnp.float32)], axis=1)
    cx = jnp.zeros((1, ROWW), jnp.float32).at[0, 75:78].set(1.0)
    md = jnp.concatenate(
        [gwt @ adr_m, jnp.zeros((F_IN, ADW - H), jnp.float32)], axis=1)
    xpe, ade = _prelude(xi, mx, cx, md)
    src, dst = edge_index[0], edge_index[1]
    wargs = (gat_b.reshape(1, C), w_ih.T, w_hh.T,
             b_ih.reshape(1, 3 * HID), b_hh.reshape(1, 3 * HID),
             p1_w, p1_b.reshape(1, 1), p2_w.T, p2_b.reshape(1, PRED))

    accA = _build_gat_sc(0)(src, dst, xpe, ade)
    accB = _build_gat_sc(1)(src, dst, xpe, ade)

    def half(acc2, q0, q1):
        accn = jnp.concatenate([acc2[0, :QUART], acc2[1, :QUART]], axis=0)
        accn = jnp.pad(accn, ((0, NPF - 2 * QUART), (0, 0)))
        xh = jnp.concatenate([xpe[q0 * QUART:(q0 + 1) * QUART],
                              xpe[q1 * QUART:(q1 + 1) * QUART]], axis=0)
        xh = jnp.pad(xh, ((0, NPF - 2 * QUART), (0, 0)))
        ah = jnp.concatenate([ade[q0 * QUART:(q0 + 1) * QUART],
                              ade[q1 * QUART:(q1 + 1) * QUART]], axis=0)
        ah = jnp.pad(ah, ((0, NPF - 2 * QUART), (0, 0)))
        return _finale(accn, xh, ah, *wargs)

    outA = half(accA, 0, 2)
    outB = half(accB, 1, 3)
    out = jnp.concatenate([outA[:QUART], outB[:QUART],
                           outA[QUART:2 * QUART], outB[QUART:2 * QUART]],
                          axis=0)
    return jnp.transpose(out.reshape(1, N, PRED), (0, 2, 1))
